# baseline (speedup 1.0000x reference)
import numpy as np
import jax
import jax.numpy as jnp
from jax.experimental import pallas as pl
from jax.experimental.pallas import tpu as pltpu
from jax.experimental.pallas import tpu_sc as plsc

_B, _C_IN, _L = 16, 64, 4096
_CODE_DIM, _K = 32, 512
_LT = 4096
_GW = 128


def _enc_body(x_ref, we_ref, be_ref, cb_ref, cbsq_ref, z_ref, idx_ref):
    xt = x_ref[0]
    z = jnp.dot(we_ref[...], xt, preferred_element_type=jnp.float32) + be_ref[...]
    z_ref[0] = z
    row_sq = jnp.sum(z * z, axis=0, keepdims=True)
    cross2 = jnp.dot(cb_ref[...], z + z, preferred_element_type=jnp.float32)
    d2 = (row_sq - cross2) + cbsq_ref[...]
    m = jnp.min(d2, axis=0, keepdims=True)
    kiota = jax.lax.broadcasted_iota(jnp.int32, d2.shape, 0)
    cand = jnp.where(d2 == m, kiota, _K)
    idx_ref[0] = jnp.min(cand, axis=0, keepdims=True)


def _dec_body(q_ref, wd_ref, bd_ref, recon_ref, zq_ref):
    zq = q_ref[0][:, :_CODE_DIM].T
    zq_ref[0] = zq
    recon_ref[0] = (
        jnp.dot(wd_ref[...], zq, preferred_element_type=jnp.float32) + bd_ref[...]
    )


def _sc_gather(codebook, idx_flat):
    n = idx_flat.shape[1]
    mesh = plsc.VectorSubcoreMesh(core_axis_name="core",
                                  subcore_axis_name="subcore")

    @pl.kernel(out_type=jax.ShapeDtypeStruct((n, 128), codebook.dtype),
               mesh=mesh)
    def k(cb_hbm, i_hbm, o_hbm):
        def body(i_vmem, o_vmem):
            pltpu.sync_copy(cb_hbm.at[i_vmem.at[0]], o_vmem)

        pltpu.emit_pipeline(
            body,
            grid=(n // _GW,),
            in_specs=[pl.BlockSpec((1, _GW), index_map=lambda i: (0, i))],
            out_specs=[pl.BlockSpec((_GW, 128), index_map=lambda i: (i, 0))],
            core_axis_name="subcore",
            dimension_semantics=(pltpu.PARALLEL,),
        )(i_hbm, o_hbm)

    return k(codebook, idx_flat)


def kernel(x, W_enc, b_enc, codebook, W_dec, b_dec):
    cb_sq = jnp.sum(codebook * codebook, axis=1)[:, None]
    grid = (_B, _L // _LT)
    z, idx = pl.pallas_call(
        _enc_body,
        grid=grid,
        in_specs=[
            pl.BlockSpec((1, _C_IN, _LT), lambda b, l: (b, 0, l)),
            pl.BlockSpec((_CODE_DIM, _C_IN), lambda b, l: (0, 0)),
            pl.BlockSpec((_CODE_DIM, 1), lambda b, l: (0, 0)),
            pl.BlockSpec((_K, _CODE_DIM), lambda b, l: (0, 0)),
            pl.BlockSpec((_K, 1), lambda b, l: (0, 0)),
        ],
        out_specs=[
            pl.BlockSpec((1, _CODE_DIM, _LT), lambda b, l: (b, 0, l)),
            pl.BlockSpec((1, 1, _LT), lambda b, l: (b, 0, l)),
        ],
        out_shape=[
            jax.ShapeDtypeStruct((_B, _CODE_DIM, _L), jnp.float32),
            jax.ShapeDtypeStruct((_B, 1, _L), jnp.int32),
        ],
    )(x, W_enc, b_enc[:, None], codebook, cb_sq)

    cb_pad = jnp.pad(codebook, ((0, 0), (0, 128 - _CODE_DIM)))
    q = _sc_gather(cb_pad, idx.reshape(1, _B * _L))

    recon, z_q = pl.pallas_call(
        _dec_body,
        grid=grid,
        in_specs=[
            pl.BlockSpec((1, _LT, 128), lambda b, l: (b, l, 0)),
            pl.BlockSpec((_C_IN, _CODE_DIM), lambda b, l: (0, 0)),
            pl.BlockSpec((_C_IN, 1), lambda b, l: (0, 0)),
        ],
        out_specs=[
            pl.BlockSpec((1, _C_IN, _LT), lambda b, l: (b, 0, l)),
            pl.BlockSpec((1, _CODE_DIM, _LT), lambda b, l: (b, 0, l)),
        ],
        out_shape=[
            jax.ShapeDtypeStruct((_B, _C_IN, _L), jnp.float32),
            jax.ShapeDtypeStruct((_B, _CODE_DIM, _L), jnp.float32),
        ],
    )(q.reshape(_B, _L, 128), W_dec, b_dec[:, None])
    return (recon, z_q, z)


# speedup vs baseline: 21.7608x; 21.7608x over previous
import numpy as np
import jax
import jax.numpy as jnp
from jax.sharding import Mesh, NamedSharding, PartitionSpec as P
from jax.experimental import pallas as pl
from jax.experimental.pallas import tpu as pltpu

_B, _C_IN, _L = 16, 64, 4096
_CODE_DIM, _K = 32, 512
_LT = 4096


def _vq_body(x_ref, we_ref, be_ref, cbaug_ref, cbt_ref, wd_ref, bd_ref,
             recon_ref, zq_ref, z_ref):
    xt = x_ref[0]
    z = jnp.dot(we_ref[...], xt, preferred_element_type=jnp.float32) + be_ref[...]
    z_ref[0] = z
    row_sq = jnp.sum(z * z, axis=0, keepdims=True)
    ones = jnp.ones((1, z.shape[1]), jnp.float32)
    z_aug = jnp.concatenate([z, ones], axis=0)
    acc = jnp.dot(cbaug_ref[...], z_aug, preferred_element_type=jnp.float32)
    d2 = row_sq + acc
    m = jnp.min(d2, axis=0, keepdims=True)
    kiota = jax.lax.broadcasted_iota(jnp.int32, d2.shape, 0)
    cand = jnp.where(d2 == m, kiota, _K)
    idx = jnp.min(cand, axis=0, keepdims=True)
    onehot = (kiota == idx).astype(jnp.float32)
    zq = jnp.dot(cbt_ref[...], onehot, preferred_element_type=jnp.float32)
    zq_ref[0] = zq
    recon_ref[0] = (
        jnp.dot(wd_ref[...], zq, preferred_element_type=jnp.float32) + bd_ref[...]
    )


def _vq_pallas(x, W_enc, b_enc2, cb_aug, cbT, W_dec, b_dec2):
    nb = x.shape[0]
    grid = (nb, _L // _LT)
    return pl.pallas_call(
        _vq_body,
        grid=grid,
        in_specs=[
            pl.BlockSpec((1, _C_IN, _LT), lambda b, l: (b, 0, l)),
            pl.BlockSpec((_CODE_DIM, _C_IN), lambda b, l: (0, 0)),
            pl.BlockSpec((_CODE_DIM, 1), lambda b, l: (0, 0)),
            pl.BlockSpec((_K, _CODE_DIM + 1), lambda b, l: (0, 0)),
            pl.BlockSpec((_CODE_DIM, _K), lambda b, l: (0, 0)),
            pl.BlockSpec((_C_IN, _CODE_DIM), lambda b, l: (0, 0)),
            pl.BlockSpec((_C_IN, 1), lambda b, l: (0, 0)),
        ],
        out_specs=[
            pl.BlockSpec((1, _C_IN, _LT), lambda b, l: (b, 0, l)),
            pl.BlockSpec((1, _CODE_DIM, _LT), lambda b, l: (b, 0, l)),
            pl.BlockSpec((1, _CODE_DIM, _LT), lambda b, l: (b, 0, l)),
        ],
        out_shape=[
            jax.ShapeDtypeStruct((nb, _C_IN, _L), jnp.float32),
            jax.ShapeDtypeStruct((nb, _CODE_DIM, _L), jnp.float32),
            jax.ShapeDtypeStruct((nb, _CODE_DIM, _L), jnp.float32),
        ],
        compiler_params=pltpu.CompilerParams(
            dimension_semantics=("parallel", "parallel"),
        ),
    )(x, W_enc, b_enc2, cb_aug, cbT, W_dec, b_dec2)


def kernel(x, W_enc, b_enc, codebook, W_dec, b_dec):
    cb_sq = jnp.sum(codebook * codebook, axis=1)[:, None]
    cb_aug = jnp.concatenate([codebook * -2.0, cb_sq], axis=1)
    args = (x, W_enc, b_enc[:, None], cb_aug, codebook.T,
            W_dec, b_dec[:, None])
    recon, z_q, z = _vq_pallas(*args)
    return (recon, z_q, z)
